# baseline (speedup 1.0000x reference)
import functools

import jax
import jax.numpy as jnp
from jax import lax
from jax.experimental import pallas as pl
from jax.experimental.pallas import tpu as pltpu
from jax.experimental.pallas import tpu_sc as plsc

N = 10000
E = 160000
D = 256
BETA = 0.1
A_SELF = 0.5

NPAD = 10240
HALFN = NPAD // 2
ACC_R = HALFN + 8
G = 16
GW = 16
EPAD = 163840
CHUNK = 128
NCH = EPAD // CHUNK
PH = 16
NCHP = NCH // PH
NCH_DEG = NCH // 16
BN = 1024
NB = NPAD // BN
NBH = NB // 2

_mesh = plsc.VectorSubcoreMesh(core_axis_name="c", subcore_axis_name="s")


def _zero_acc(acc):
    zeros16 = jnp.zeros((16,), jnp.float32)

    def zbody(i, carry):
        acc[pl.ds(i * 16, 16)] = zeros16
        return carry

    lax.fori_loop(0, ACC_R * GW // 16, zbody, 0)


@functools.partial(
    pl.kernel,
    out_type=jax.ShapeDtypeStruct((32 * HALFN * GW,), jnp.float32),
    mesh=_mesh,
    scratch_types=[
        pltpu.VMEM((NCH_DEG, CHUNK), jnp.int32),
    ],
    compiler_params=pltpu.CompilerParams(needs_layout_passes=False),
)
def _deg_pass(dst_h, out, dst_v):
    c = lax.axis_index("c")
    s = lax.axis_index("s")
    w = s * 2 + c
    pltpu.sync_copy(dst_h.at[pl.ds(c * NCH + s * NCH_DEG, NCH_DEG)], dst_v)
    ones16 = jnp.ones((16,), jnp.float32)
    iota16 = lax.iota(jnp.int32, 16)

    def scoped(acc):
        _zero_acc(acc)

        def body(j, carry):
            for u in range(CHUNK // 16):
                row16 = dst_v[j, pl.ds(u * 16, 16)]
                plsc.addupdate_scatter(acc, [row16 * GW + iota16], ones16)
            return carry

        lax.fori_loop(0, NCH_DEG, body, 0)
        pltpu.sync_copy(acc.at[pl.ds(0, HALFN * GW)],
                        out.at[pl.ds(w * HALFN * GW, HALFN * GW)])

    pl.run_scoped(scoped, pltpu.VMEM((ACC_R * GW,), jnp.float32))


NCH_A = NCH // 32


@functools.partial(
    pl.kernel,
    out_type=jax.ShapeDtypeStruct((G * EPAD * GW,), jnp.float32),
    mesh=_mesh,
    scratch_types=[
        pltpu.VMEM((NCH_A, CHUNK), jnp.int32),
        pltpu.VMEM((CHUNK, D), jnp.float32),
        pltpu.VMEM((CHUNK, D), jnp.float32),
        pltpu.VMEM((G * CHUNK * GW,), jnp.float32),
        pltpu.SemaphoreType.DMA,
        pltpu.SemaphoreType.DMA,
    ],
    compiler_params=pltpu.CompilerParams(needs_layout_passes=False),
)
def _gather_pass(hp, src2, out, src_v, buf0, buf1, gbuf, sem0, sem1):
    c = lax.axis_index("c")
    s = lax.axis_index("s")
    w = s * 2 + c
    pltpu.sync_copy(src2.at[pl.ds(w * NCH_A, NCH_A)], src_v)

    bufs = (buf0, buf1)
    sems = (sem0, sem1)

    def start(j, b):
        pltpu.make_async_copy(hp.at[src_v.at[j]], bufs[b], sems[b]).start()

    def wait(j, b):
        pltpu.make_async_copy(hp.at[src_v.at[j]], bufs[b], sems[b]).wait()

    def emit(j, b):
        rows = bufs[b]

        def ebody(e, carry):
            for g in range(G):
                gbuf[pl.ds(g * (CHUNK * GW) + e * GW, GW)] = (
                    rows[e, pl.ds(g * GW, GW)])
            return carry

        lax.fori_loop(0, CHUNK, ebody, 0)
        for g in range(G):
            pltpu.sync_copy(
                gbuf.at[pl.ds(g * (CHUNK * GW), CHUNK * GW)],
                out.at[pl.ds(
                    (g * EPAD + w * (NCH_A * CHUNK) + j * CHUNK) * GW,
                    CHUNK * GW)])

    start(0, 0)

    def body(i, carry):
        j0 = 2 * i
        j1 = j0 + 1
        start(j1, 1)
        wait(j0, 0)
        emit(j0, 0)

        @pl.when(j1 + 1 < NCH_A)
        def _():
            start(j1 + 1, 0)

        wait(j1, 1)
        emit(j1, 1)
        return carry

    lax.fori_loop(0, NCH_A // 2, body, 0)


@functools.partial(
    pl.kernel,
    out_type=jax.ShapeDtypeStruct((32 * HALFN * GW,), jnp.float32),
    mesh=_mesh,
    scratch_types=[
        pltpu.VMEM((NCHP, CHUNK), jnp.int32),
        pltpu.VMEM((CHUNK * GW,), jnp.float32),
        pltpu.VMEM((CHUNK * GW,), jnp.float32),
        pltpu.VMEM((ACC_R * GW,), jnp.float32),
        pltpu.SemaphoreType.DMA,
        pltpu.SemaphoreType.DMA,
    ],
    compiler_params=pltpu.CompilerParams(needs_layout_passes=False),
)
def _acc_pass(msg, dst_h, out, dst_v, buf0, buf1, acc, sem0, sem1):
    c = lax.axis_index("c")
    s = lax.axis_index("s")
    w = s * 2 + c
    _zero_acc(acc)

    bufs = (buf0, buf1)
    sems = (sem0, sem1)

    def start(p, j, b):
        pltpu.make_async_copy(
            msg.at[pl.ds((s * EPAD + (p * NCHP + j) * CHUNK) * GW,
                         CHUNK * GW)],
            bufs[b], sems[b]).start()

    def wait(p, j, b):
        pltpu.make_async_copy(
            msg.at[pl.ds((s * EPAD + (p * NCHP + j) * CHUNK) * GW,
                         CHUNK * GW)],
            bufs[b], sems[b]).wait()

    iota16 = lax.iota(jnp.int32, 16)

    def accumulate(j, b):
        rows = bufs[b]

        def ubody(u, carry):
            row16 = dst_v[j, pl.ds(u * 16, 16)]
            rbase16 = row16 * GW
            edge16 = u * 16 + iota16
            ebase16 = edge16 * GW
            for k in range(GW):
                vals = plsc.load_gather(rows, [ebase16 + k])
                plsc.addupdate_scatter(acc, [rbase16 + k], vals)
            return carry

        lax.fori_loop(0, CHUNK // 16, ubody, 0)

    for p in range(PH):
        pltpu.sync_copy(dst_h.at[pl.ds((c * PH + p) * NCHP, NCHP)], dst_v)
        start(p, 0, 0)

        def body(i, carry):
            j0 = 2 * i
            j1 = j0 + 1
            start(p, j1, 1)
            wait(p, j0, 0)
            accumulate(j0, 0)

            @pl.when(j1 + 1 < NCHP)
            def _():
                start(p, j1 + 1, 0)

            wait(p, j1, 1)
            accumulate(j1, 1)
            return carry

        lax.fori_loop(0, NCHP // 2, body, 0)
    pltpu.sync_copy(acc.at[pl.ds(0, HALFN * GW)],
                    out.at[pl.ds(w * HALFN * GW, HALFN * GW)])


def _deg_spec(s):
    return pl.BlockSpec(
        (BN, GW), lambda i, s=s: ((s * 2 + i // NBH) * NBH + i % NBH, 0))


def _agg_spec(g):
    return pl.BlockSpec(
        (BN, GW), lambda i, g=g: ((g * 2 + i // NBH) * NBH + i % NBH, 0))


def _row_spec(width):
    return pl.BlockSpec((BN, width), lambda i: (i, 0))


def _full_spec(h, width):
    return pl.BlockSpec((h, width), lambda i: (0, 0))


def _first_body(x_ref, *rest):
    deg_refs = rest[:G]
    w_ref = rest[G]
    hp_ref, dinv_ref = rest[G + 1], rest[G + 2]
    cnt = sum(jnp.sum(r[...], axis=1, keepdims=True) for r in deg_refs)
    dinv = lax.rsqrt(cnt + 2.0)
    dinv_ref[...] = jnp.broadcast_to(dinv, (BN, 8))
    hp_ref[...] = jnp.dot(x_ref[...], w_ref[...],
                          preferred_element_type=jnp.float32) * dinv


def _mid_body(x_ref, dinv_ref, *rest):
    agg_refs = rest[:G]
    hp_ref = rest[G]
    b_ref, w_ref = rest[G + 1], rest[G + 2]
    o_ref = rest[G + 3]
    dinv = dinv_ref[...][:, :1]
    agg = jnp.concatenate([r[...] for r in agg_refs], axis=1)
    conv = dinv * (agg + 2.0 * hp_ref[...]) + b_ref[...]
    z = BETA * x_ref[...] + (1.0 - BETA) * conv
    o_ref[...] = jnp.dot(z, w_ref[...],
                         preferred_element_type=jnp.float32) * dinv


def _final_body(x_ref, dinv_ref, *rest):
    agg_refs = rest[:G]
    hp_ref = rest[G]
    b_ref = rest[G + 1]
    o_ref = rest[G + 2]
    dinv = dinv_ref[...][:, :1]
    agg = jnp.concatenate([r[...] for r in agg_refs], axis=1)
    conv = dinv * (agg + 2.0 * hp_ref[...]) + b_ref[...]
    o_ref[...] = A_SELF * x_ref[...] + (1.0 - A_SELF) * conv


_HP_OUT = jax.ShapeDtypeStruct((NPAD, D), jnp.float32)

_first_tc = pl.pallas_call(
    _first_body,
    grid=(NB,),
    in_specs=[_row_spec(D)] + [_deg_spec(s) for s in range(G)]
    + [_full_spec(D, D)],
    out_specs=[_row_spec(D), _row_spec(8)],
    out_shape=[_HP_OUT, jax.ShapeDtypeStruct((NPAD, 8), jnp.float32)],
)

_mid_tc = pl.pallas_call(
    _mid_body,
    grid=(NB,),
    in_specs=[_row_spec(D), _row_spec(8)]
    + [_agg_spec(g) for g in range(G)]
    + [_row_spec(D), _full_spec(1, D), _full_spec(D, D)],
    out_specs=_row_spec(D),
    out_shape=_HP_OUT,
)

_final_tc = pl.pallas_call(
    _final_body,
    grid=(NB,),
    in_specs=[_row_spec(D), _row_spec(8)]
    + [_agg_spec(g) for g in range(G)]
    + [_row_spec(D), _full_spec(1, D)],
    out_specs=_row_spec(D),
    out_shape=jax.ShapeDtypeStruct((NPAD, D), jnp.float32),
)


def kernel(x, edge_index, W1, b1, W2, b2, W3, b3, W4, b4):
    src = edge_index[0]
    dst = edge_index[1]
    x_pad = jnp.pad(x, ((0, NPAD - N), (0, 0)))
    src_pad = jnp.concatenate([src, jnp.zeros((EPAD - E,), jnp.int32)])
    dst_pad = jnp.concatenate([dst, jnp.full((EPAD - E,), N, jnp.int32)])
    src2 = src_pad.reshape(NCH, CHUNK)
    dst2 = dst_pad.reshape(NCH, CHUNK)
    dummy = jnp.int32(HALFN)
    dst_h = jnp.concatenate(
        [jnp.where(dst2 < HALFN, dst2, dummy),
         jnp.where(dst2 >= HALFN, dst2 - HALFN, dummy)], axis=0)

    degbuf = _deg_pass(dst_h).reshape(32 * HALFN, GW)

    def layer_agg(hp):
        msg = _gather_pass(hp, src2)
        return _acc_pass(msg, dst_h).reshape(32 * HALFN, GW)

    hp, dinv8 = _first_tc(x_pad, *([degbuf] * G), W1)
    for (Wn, bp) in ((W2, b1), (W3, b2), (W4, b3)):
        agg = layer_agg(hp)
        hp = _mid_tc(x_pad, dinv8, *([agg] * G), hp, bp.reshape(1, D), Wn)
    agg = layer_agg(hp)
    out = _final_tc(x_pad, dinv8, *([agg] * G), hp, b4.reshape(1, D))
    return out[:N]


# speedup vs baseline: 1.0106x; 1.0106x over previous
import functools

import jax
import jax.numpy as jnp
from jax import lax
from jax.experimental import pallas as pl
from jax.experimental.pallas import tpu as pltpu
from jax.experimental.pallas import tpu_sc as plsc

N = 10000
E = 160000
D = 256
BETA = 0.1
A_SELF = 0.5

NPAD = 10240
HALFN = NPAD // 2
ACC_R = HALFN + 8
G = 16
GW = 16
EPAD = 163840
CHUNK = 128
NCH = EPAD // CHUNK
PH = 16
NCHP = NCH // PH
NCH_DEG = NCH // 16
BN = 1024
NB = NPAD // BN
NBH = NB // 2

_mesh = plsc.VectorSubcoreMesh(core_axis_name="c", subcore_axis_name="s")


def _zero_acc(acc):
    zeros16 = jnp.zeros((16,), jnp.float32)

    def zbody(i, carry):
        acc[pl.ds(i * 16, 16)] = zeros16
        return carry

    lax.fori_loop(0, ACC_R * GW // 16, zbody, 0)


@functools.partial(
    pl.kernel,
    out_type=jax.ShapeDtypeStruct((32 * HALFN * GW,), jnp.float32),
    mesh=_mesh,
    scratch_types=[
        pltpu.VMEM((NCH_DEG, CHUNK), jnp.int32),
    ],
    compiler_params=pltpu.CompilerParams(needs_layout_passes=False),
)
def _deg_pass(dst_h, out, dst_v):
    c = lax.axis_index("c")
    s = lax.axis_index("s")
    w = s * 2 + c
    pltpu.sync_copy(dst_h.at[pl.ds(c * NCH + s * NCH_DEG, NCH_DEG)], dst_v)
    ones16 = jnp.ones((16,), jnp.float32)
    iota16 = lax.iota(jnp.int32, 16)

    def scoped(acc):
        _zero_acc(acc)

        def body(j, carry):
            for u in range(CHUNK // 16):
                row16 = dst_v[j, pl.ds(u * 16, 16)]
                plsc.addupdate_scatter(acc, [row16 * GW + iota16], ones16)
            return carry

        lax.fori_loop(0, NCH_DEG, body, 0)
        pltpu.sync_copy(acc.at[pl.ds(0, HALFN * GW)],
                        out.at[pl.ds(w * HALFN * GW, HALFN * GW)])

    pl.run_scoped(scoped, pltpu.VMEM((ACC_R * GW,), jnp.float32))


NCH_A = NCH // 32


@functools.partial(
    pl.kernel,
    out_type=jax.ShapeDtypeStruct((G * EPAD * GW,), jnp.float32),
    mesh=_mesh,
    scratch_types=[
        pltpu.VMEM((NCH_A, CHUNK), jnp.int32),
        pltpu.VMEM((CHUNK, D), jnp.float32),
        pltpu.VMEM((CHUNK, D), jnp.float32),
        pltpu.VMEM((G * CHUNK * GW,), jnp.float32),
        pltpu.SemaphoreType.DMA,
        pltpu.SemaphoreType.DMA,
    ],
    compiler_params=pltpu.CompilerParams(needs_layout_passes=False),
)
def _gather_pass(hp, src2, out, src_v, buf0, buf1, gbuf, sem0, sem1):
    c = lax.axis_index("c")
    s = lax.axis_index("s")
    w = s * 2 + c
    pltpu.sync_copy(src2.at[pl.ds(w * NCH_A, NCH_A)], src_v)

    bufs = (buf0, buf1)
    sems = (sem0, sem1)

    def start(j, b):
        pltpu.make_async_copy(hp.at[src_v.at[j]], bufs[b], sems[b]).start()

    def wait(j, b):
        pltpu.make_async_copy(hp.at[src_v.at[j]], bufs[b], sems[b]).wait()

    def emit(j, b):
        rows = bufs[b]

        def ebody(e, carry):
            for g in range(G):
                gbuf[pl.ds(g * (CHUNK * GW) + e * GW, GW)] = (
                    rows[e, pl.ds(g * GW, GW)])
            return carry

        lax.fori_loop(0, CHUNK, ebody, 0)
        pltpu.sync_copy(
            gbuf, out.at[pl.ds((w * NCH_A + j) * (G * CHUNK * GW),
                               G * CHUNK * GW)])

    start(0, 0)

    def body(i, carry):
        j0 = 2 * i
        j1 = j0 + 1
        start(j1, 1)
        wait(j0, 0)
        emit(j0, 0)

        @pl.when(j1 + 1 < NCH_A)
        def _():
            start(j1 + 1, 0)

        wait(j1, 1)
        emit(j1, 1)
        return carry

    lax.fori_loop(0, NCH_A // 2, body, 0)


@functools.partial(
    pl.kernel,
    out_type=jax.ShapeDtypeStruct((32 * HALFN * GW,), jnp.float32),
    mesh=_mesh,
    scratch_types=[
        pltpu.VMEM((NCHP, CHUNK), jnp.int32),
        pltpu.VMEM((CHUNK * GW,), jnp.float32),
        pltpu.VMEM((CHUNK * GW,), jnp.float32),
        pltpu.VMEM((ACC_R * GW,), jnp.float32),
        pltpu.SemaphoreType.DMA,
        pltpu.SemaphoreType.DMA,
    ],
    compiler_params=pltpu.CompilerParams(needs_layout_passes=False),
)
def _acc_pass(msg, dst_h, out, dst_v, buf0, buf1, acc, sem0, sem1):
    c = lax.axis_index("c")
    s = lax.axis_index("s")
    w = s * 2 + c
    _zero_acc(acc)

    bufs = (buf0, buf1)
    sems = (sem0, sem1)

    def start(p, j, b):
        pltpu.make_async_copy(
            msg.at[pl.ds(((p * NCHP + j) * G + s) * (CHUNK * GW),
                         CHUNK * GW)],
            bufs[b], sems[b]).start()

    def wait(p, j, b):
        pltpu.make_async_copy(
            msg.at[pl.ds(((p * NCHP + j) * G + s) * (CHUNK * GW),
                         CHUNK * GW)],
            bufs[b], sems[b]).wait()

    iota16 = lax.iota(jnp.int32, 16)

    def accumulate(j, b):
        rows = bufs[b]

        def ubody(u, carry):
            row16 = dst_v[j, pl.ds(u * 16, 16)]
            rbase16 = row16 * GW
            edge16 = u * 16 + iota16
            ebase16 = edge16 * GW
            for k in range(GW):
                vals = plsc.load_gather(rows, [ebase16 + k])
                plsc.addupdate_scatter(acc, [rbase16 + k], vals)
            return carry

        lax.fori_loop(0, CHUNK // 16, ubody, 0)

    for p in range(PH):
        pltpu.sync_copy(dst_h.at[pl.ds((c * PH + p) * NCHP, NCHP)], dst_v)
        start(p, 0, 0)

        def body(i, carry):
            j0 = 2 * i
            j1 = j0 + 1
            start(p, j1, 1)
            wait(p, j0, 0)
            accumulate(j0, 0)

            @pl.when(j1 + 1 < NCHP)
            def _():
                start(p, j1 + 1, 0)

            wait(p, j1, 1)
            accumulate(j1, 1)
            return carry

        lax.fori_loop(0, NCHP // 2, body, 0)
    pltpu.sync_copy(acc.at[pl.ds(0, HALFN * GW)],
                    out.at[pl.ds(w * HALFN * GW, HALFN * GW)])


def _deg_spec(s):
    return pl.BlockSpec(
        (BN, GW), lambda i, s=s: ((s * 2 + i // NBH) * NBH + i % NBH, 0))


def _agg_spec(g):
    return pl.BlockSpec(
        (BN, GW), lambda i, g=g: ((g * 2 + i // NBH) * NBH + i % NBH, 0))


def _row_spec(width):
    return pl.BlockSpec((BN, width), lambda i: (i, 0))


def _full_spec(h, width):
    return pl.BlockSpec((h, width), lambda i: (0, 0))


def _first_body(x_ref, *rest):
    deg_refs = rest[:G]
    w_ref = rest[G]
    hp_ref, dinv_ref = rest[G + 1], rest[G + 2]
    cnt = sum(jnp.sum(r[...], axis=1, keepdims=True) for r in deg_refs)
    dinv = lax.rsqrt(cnt + 2.0)
    dinv_ref[...] = jnp.broadcast_to(dinv, (BN, 8))
    hp_ref[...] = jnp.dot(x_ref[...], w_ref[...],
                          preferred_element_type=jnp.float32) * dinv


def _mid_body(x_ref, dinv_ref, *rest):
    agg_refs = rest[:G]
    hp_ref = rest[G]
    b_ref, w_ref = rest[G + 1], rest[G + 2]
    o_ref = rest[G + 3]
    dinv = dinv_ref[...][:, :1]
    agg = jnp.concatenate([r[...] for r in agg_refs], axis=1)
    conv = dinv * (agg + 2.0 * hp_ref[...]) + b_ref[...]
    z = BETA * x_ref[...] + (1.0 - BETA) * conv
    o_ref[...] = jnp.dot(z, w_ref[...],
                         preferred_element_type=jnp.float32) * dinv


def _final_body(x_ref, dinv_ref, *rest):
    agg_refs = rest[:G]
    hp_ref = rest[G]
    b_ref = rest[G + 1]
    o_ref = rest[G + 2]
    dinv = dinv_ref[...][:, :1]
    agg = jnp.concatenate([r[...] for r in agg_refs], axis=1)
    conv = dinv * (agg + 2.0 * hp_ref[...]) + b_ref[...]
    o_ref[...] = A_SELF * x_ref[...] + (1.0 - A_SELF) * conv


_HP_OUT = jax.ShapeDtypeStruct((NPAD, D), jnp.float32)

_first_tc = pl.pallas_call(
    _first_body,
    grid=(NB,),
    in_specs=[_row_spec(D)] + [_deg_spec(s) for s in range(G)]
    + [_full_spec(D, D)],
    out_specs=[_row_spec(D), _row_spec(8)],
    out_shape=[_HP_OUT, jax.ShapeDtypeStruct((NPAD, 8), jnp.float32)],
)

_mid_tc = pl.pallas_call(
    _mid_body,
    grid=(NB,),
    in_specs=[_row_spec(D), _row_spec(8)]
    + [_agg_spec(g) for g in range(G)]
    + [_row_spec(D), _full_spec(1, D), _full_spec(D, D)],
    out_specs=_row_spec(D),
    out_shape=_HP_OUT,
)

_final_tc = pl.pallas_call(
    _final_body,
    grid=(NB,),
    in_specs=[_row_spec(D), _row_spec(8)]
    + [_agg_spec(g) for g in range(G)]
    + [_row_spec(D), _full_spec(1, D)],
    out_specs=_row_spec(D),
    out_shape=jax.ShapeDtypeStruct((NPAD, D), jnp.float32),
)


def kernel(x, edge_index, W1, b1, W2, b2, W3, b3, W4, b4):
    src = edge_index[0]
    dst = edge_index[1]
    x_pad = jnp.pad(x, ((0, NPAD - N), (0, 0)))
    src_pad = jnp.concatenate([src, jnp.zeros((EPAD - E,), jnp.int32)])
    dst_pad = jnp.concatenate([dst, jnp.full((EPAD - E,), N, jnp.int32)])
    src2 = src_pad.reshape(NCH, CHUNK)
    dst2 = dst_pad.reshape(NCH, CHUNK)
    dummy = jnp.int32(HALFN)
    dst_h = jnp.concatenate(
        [jnp.where(dst2 < HALFN, dst2, dummy),
         jnp.where(dst2 >= HALFN, dst2 - HALFN, dummy)], axis=0)

    degbuf = _deg_pass(dst_h).reshape(32 * HALFN, GW)

    def layer_agg(hp):
        msg = _gather_pass(hp, src2)
        return _acc_pass(msg, dst_h).reshape(32 * HALFN, GW)

    hp, dinv8 = _first_tc(x_pad, *([degbuf] * G), W1)
    for (Wn, bp) in ((W2, b1), (W3, b2), (W4, b3)):
        agg = layer_agg(hp)
        hp = _mid_tc(x_pad, dinv8, *([agg] * G), hp, bp.reshape(1, D), Wn)
    agg = layer_agg(hp)
    out = _final_tc(x_pad, dinv8, *([agg] * G), hp, b4.reshape(1, D))
    return out[:N]
